# baseline (speedup 1.0000x reference)
import functools

import jax
import jax.numpy as jnp
from jax import lax
from jax.experimental import pallas as pl
from jax.experimental.pallas import tpu as pltpu
from jax.experimental.pallas import tpu_sc as plsc

N = 4096
D = 1024
F = 4096
E = 8
TILE = 256
PADDED = 10240
NT = PADDED // TILE
FT = 512
NJ = F // FT
NW = 32
CHT = N // NW
SUB = 32


def _router_body(x_ref, gw_ref, gb_ref, lg_ref, pos_ref, te_ref, wv_ref):
    x = x_ref[...]
    lg = jnp.dot(x, gw_ref[...], preferred_element_type=jnp.float32) + gb_ref[...]
    lg_ref[...] = lg
    mx = jnp.max(lg, axis=1, keepdims=True)
    ex = jnp.exp(lg - mx)
    sm = ex / jnp.sum(ex, axis=1, keepdims=True)
    io = lax.broadcasted_iota(jnp.int32, sm.shape, 1)
    m1 = jnp.max(sm, axis=1, keepdims=True)
    i1 = jnp.min(jnp.where(sm == m1, io, E), axis=1, keepdims=True)
    oh1 = (io == i1).astype(jnp.float32)
    sm2 = jnp.where(io == i1, -1.0, sm)
    m2 = jnp.max(sm2, axis=1, keepdims=True)
    i2 = jnp.min(jnp.where(sm2 == m2, io, E), axis=1, keepdims=True)
    oh2 = (io == i2).astype(jnp.float32)
    den = m1 + m2
    wv_ref[...] = jnp.concatenate([m1 / den, m2 / den], axis=1)
    c = oh1 + oh2
    nb = N // 128
    tri = (lax.broadcasted_iota(jnp.int32, (128, 128), 0)
           > lax.broadcasted_iota(jnp.int32, (128, 128), 1)).astype(jnp.float32)
    withins = [jnp.dot(tri, lax.slice(c, (b * 128, 0), ((b + 1) * 128, E)),
                       preferred_element_type=jnp.float32) for b in range(nb)]
    bs = jnp.concatenate(
        [lax.slice(w, (127, 0), (128, E))
         + lax.slice(c, (b * 128 + 127, 0), (b * 128 + 128, E))
         for b, w in enumerate(withins)], axis=0)
    tri_b = (lax.broadcasted_iota(jnp.int32, (nb, nb), 0)
             > lax.broadcasted_iota(jnp.int32, (nb, nb), 1)).astype(jnp.float32)
    boffs = jnp.dot(tri_b, bs, preferred_element_type=jnp.float32)
    excl = jnp.concatenate(
        [w + lax.slice(boffs, (b, 0), (b + 1, E))
         for b, w in enumerate(withins)], axis=0)
    counts = (lax.slice(boffs, (nb - 1, 0), (nb, E))
              + lax.slice(bs, (nb - 1, 0), (nb, E)))
    padded = jnp.floor((counts + (TILE - 1.0)) * (1.0 / TILE)) * TILE
    m8 = (lax.broadcasted_iota(jnp.int32, (E, E), 0)
          < lax.broadcasted_iota(jnp.int32, (E, E), 1)).astype(jnp.float32)
    offs = jnp.dot(padded, m8, preferred_element_type=jnp.float32)
    posf = offs + excl
    p0 = jnp.sum(oh1 * posf, axis=1, keepdims=True)
    p1 = jnp.sum(oh2 * posf, axis=1, keepdims=True)
    pos_ref[...] = jnp.concatenate([p0, p1], axis=1).astype(jnp.int32)
    it = lax.broadcasted_iota(jnp.int32, (48, E), 0).astype(jnp.float32) * TILE
    te = jnp.sum((offs <= it).astype(jnp.int32), axis=1, keepdims=True) - 1
    te_ref[...] = jnp.clip(te, 0, E - 1)


def _router(flat, gate_w, gate_b):
    return pl.pallas_call(
        _router_body,
        out_shape=[
            jax.ShapeDtypeStruct((N, E), jnp.float32),
            jax.ShapeDtypeStruct((N, 2), jnp.int32),
            jax.ShapeDtypeStruct((48, 1), jnp.int32),
            jax.ShapeDtypeStruct((N, 2), jnp.float32),
        ],
    )(flat, gate_w, gate_b)


def _dispatch_body(flat_hbm, posr_hbm, wrows_hbm, xs_hbm, wtab_hbm,
                   idx_v, xbuf, wrv, sem0, sem1, semw):
    wid = lax.axis_index("s") * 2 + lax.axis_index("c")
    pltpu.sync_copy(posr_hbm.at[wid], idx_v)
    pltpu.sync_copy(wrows_hbm.at[wid], wrv)
    for c in range(CHT // SUB):
        pltpu.sync_copy(flat_hbm.at[pl.ds(wid * CHT + c * SUB, SUB)], xbuf)
        cp0 = pltpu.async_copy(xbuf, xs_hbm.at[idx_v.at[c]], sem0)
        cp1 = pltpu.async_copy(xbuf, xs_hbm.at[idx_v.at[4 + c]], sem1)
        cw0 = pltpu.async_copy(wrv.at[c], wtab_hbm.at[idx_v.at[c]], semw)
        cw1 = pltpu.async_copy(wrv.at[4 + c], wtab_hbm.at[idx_v.at[4 + c]], semw)
        cp0.wait()
        cp1.wait()
        cw0.wait()
        cw1.wait()


def _dispatch(flat, posr, wrows):
    mesh = plsc.VectorSubcoreMesh(core_axis_name="c", subcore_axis_name="s")
    return pl.kernel(
        _dispatch_body,
        out_type=[
            jax.ShapeDtypeStruct((PADDED, D), jnp.float32),
            jax.ShapeDtypeStruct((PADDED, 128), jnp.float32),
        ],
        mesh=mesh,
        scratch_types=[
            pltpu.VMEM((8, SUB), jnp.int32),
            pltpu.VMEM((SUB, D), jnp.float32),
            pltpu.VMEM((8, SUB, 128), jnp.float32),
            pltpu.SemaphoreType.DMA,
            pltpu.SemaphoreType.DMA,
            pltpu.SemaphoreType.DMA,
        ],
    )(flat, posr, wrows)


def _mlp_body(te_ref, x_ref, w1_hbm, b1_ref, w2_hbm, b2_ref, wt_ref, y_ref,
              w1v, w2v, sem1, sem2, sem3, sem4):
    i = pl.program_id(0)
    e = te_ref[i]
    eprev = te_ref[jnp.maximum(i - 1, 0)]
    change = jnp.logical_or(i == 0, e != eprev)
    q = F // 4
    sems = [sem1, sem2, sem3, sem4]

    def w_copies(k):
        sl = pl.ds(k * q, q)
        return (pltpu.make_async_copy(w1_hbm.at[e, :, sl], w1v.at[:, sl],
                                      sems[k]),
                pltpu.make_async_copy(w2_hbm.at[e, sl, :], w2v.at[sl, :],
                                      sems[k]))

    @pl.when(change)
    def _():
        for k in range(4):
            c1, c2 = w_copies(k)
            c1.start()
            c2.start()

    x = x_ref[...]
    acc = None
    for j in range(NJ):
        if j % 2 == 0:
            @pl.when(change)
            def _():
                c1, c2 = w_copies(j // 2)
                c1.wait()
                c2.wait()
        sl = pl.ds(j * FT, FT)
        h = jnp.maximum(
            jnp.dot(x, w1v[:, sl], preferred_element_type=jnp.float32)
            + b1_ref[0, :, sl], 0.0)
        part = jnp.dot(h, w2v[sl, :], preferred_element_type=jnp.float32)
        acc = part if acc is None else acc + part
    y_ref[...] = (acc + b2_ref[0]) * wt_ref[:, 0:1]


def _mlp(texp, xs, w1, b1, w2, b2, wtab):
    grid_spec = pltpu.PrefetchScalarGridSpec(
        num_scalar_prefetch=1,
        grid=(NT,),
        in_specs=[
            pl.BlockSpec((TILE, D), lambda i, s: (i, 0)),
            pl.BlockSpec(memory_space=pl.ANY),
            pl.BlockSpec((1, 1, F), lambda i, s: (s[i], 0, 0)),
            pl.BlockSpec(memory_space=pl.ANY),
            pl.BlockSpec((1, 1, D), lambda i, s: (s[i], 0, 0)),
            pl.BlockSpec((TILE, 128), lambda i, s: (i, 0)),
        ],
        out_specs=pl.BlockSpec((TILE, D), lambda i, s: (i, 0)),
        scratch_shapes=[
            pltpu.VMEM((D, F), jnp.float32),
            pltpu.VMEM((F, D), jnp.float32),
            pltpu.SemaphoreType.DMA,
            pltpu.SemaphoreType.DMA,
            pltpu.SemaphoreType.DMA,
            pltpu.SemaphoreType.DMA,
        ],
    )
    return pl.pallas_call(
        _mlp_body,
        grid_spec=grid_spec,
        out_shape=jax.ShapeDtypeStruct((PADDED, D), jnp.float32),
        compiler_params=pltpu.CompilerParams(
            dimension_semantics=("arbitrary",),
            vmem_limit_bytes=60 * 1024 * 1024),
    )(texp, xs, w1, b1.reshape(E, 1, F), w2, b2.reshape(E, 1, D), wtab)


def _combine_body(ys_hbm, posr_hbm, out_hbm,
                  idx_v, y0, y1, ob, sem0, sem1):
    wid = lax.axis_index("s") * 2 + lax.axis_index("c")
    pltpu.sync_copy(posr_hbm.at[wid], idx_v)
    for c in range(CHT // SUB):
        g0 = pltpu.async_copy(ys_hbm.at[idx_v.at[c]], y0, sem0)
        g1 = pltpu.async_copy(ys_hbm.at[idx_v.at[4 + c]], y1, sem1)
        g0.wait()
        g1.wait()

        def tok_body(t, _):
            for q in range(D // 16):
                sl = pl.ds(16 * q, 16)
                ob[t, sl] = y0[t, sl] + y1[t, sl]
            return 0

        lax.fori_loop(0, SUB, tok_body, 0)
        pltpu.sync_copy(ob, out_hbm.at[pl.ds(wid * CHT + c * SUB, SUB)])


def _combine(ys, posr):
    mesh = plsc.VectorSubcoreMesh(core_axis_name="c", subcore_axis_name="s")
    return pl.kernel(
        _combine_body,
        out_type=jax.ShapeDtypeStruct((N, D), jnp.float32),
        mesh=mesh,
        scratch_types=[
            pltpu.VMEM((8, SUB), jnp.int32),
            pltpu.VMEM((SUB, D), jnp.float32),
            pltpu.VMEM((SUB, D), jnp.float32),
            pltpu.VMEM((SUB, D), jnp.float32),
            pltpu.SemaphoreType.DMA,
            pltpu.SemaphoreType.DMA,
        ],
    )(ys, posr)


def kernel(hidden_states, gate_w, gate_b, w1, b1, w2, b2):
    seq, bsz, d = hidden_states.shape
    flat = hidden_states.reshape(-1, d)
    logits, pos2, te48, wv = _router(flat, gate_w, gate_b.reshape(1, E))
    texp = te48.reshape(-1)[:NT]
    posr = (pos2.T.reshape(2, NW, CHT // SUB, SUB)
            .transpose(1, 0, 2, 3).reshape(NW, 8, SUB))
    wrows = jnp.broadcast_to(
        (wv.T.reshape(2, NW, CHT // SUB, SUB)
         .transpose(1, 0, 2, 3).reshape(NW, 8, SUB))[..., None],
        (NW, 8, SUB, 128))
    xs, wtab = _dispatch(flat, posr, wrows)
    ys = _mlp(texp, xs, w1, b1, w2, b2, wtab)
    final = _combine(ys, posr)
    return final.reshape(seq, bsz, d), logits


# speedup vs baseline: 1.0111x; 1.0111x over previous
import functools

import jax
import jax.numpy as jnp
from jax import lax
from jax.experimental import pallas as pl
from jax.experimental.pallas import tpu as pltpu
from jax.experimental.pallas import tpu_sc as plsc

N = 4096
D = 1024
F = 4096
E = 8
TILE = 256
PADDED = 10240
NT = PADDED // TILE
FT = 512
NJ = F // FT
NW = 32
CHT = N // NW
SUB = 32


def _router_body(x_ref, gw_ref, gb_ref, lg_ref, pos_ref, te_ref, wv_ref):
    x = x_ref[...]
    lg = jnp.dot(x, gw_ref[...], preferred_element_type=jnp.float32) + gb_ref[...]
    lg_ref[...] = lg
    mx = jnp.max(lg, axis=1, keepdims=True)
    ex = jnp.exp(lg - mx)
    sm = ex / jnp.sum(ex, axis=1, keepdims=True)
    io = lax.broadcasted_iota(jnp.int32, sm.shape, 1)
    m1 = jnp.max(sm, axis=1, keepdims=True)
    i1 = jnp.min(jnp.where(sm == m1, io, E), axis=1, keepdims=True)
    oh1 = (io == i1).astype(jnp.float32)
    sm2 = jnp.where(io == i1, -1.0, sm)
    m2 = jnp.max(sm2, axis=1, keepdims=True)
    i2 = jnp.min(jnp.where(sm2 == m2, io, E), axis=1, keepdims=True)
    oh2 = (io == i2).astype(jnp.float32)
    den = m1 + m2
    wv_ref[...] = jnp.concatenate([m1 / den, m2 / den], axis=1)
    c = oh1 + oh2
    nb = N // 128
    tri = (lax.broadcasted_iota(jnp.int32, (128, 128), 0)
           > lax.broadcasted_iota(jnp.int32, (128, 128), 1)).astype(jnp.float32)
    withins = [jnp.dot(tri, lax.slice(c, (b * 128, 0), ((b + 1) * 128, E)),
                       preferred_element_type=jnp.float32) for b in range(nb)]
    bs = jnp.concatenate(
        [lax.slice(w, (127, 0), (128, E))
         + lax.slice(c, (b * 128 + 127, 0), (b * 128 + 128, E))
         for b, w in enumerate(withins)], axis=0)
    tri_b = (lax.broadcasted_iota(jnp.int32, (nb, nb), 0)
             > lax.broadcasted_iota(jnp.int32, (nb, nb), 1)).astype(jnp.float32)
    boffs = jnp.dot(tri_b, bs, preferred_element_type=jnp.float32)
    excl = jnp.concatenate(
        [w + lax.slice(boffs, (b, 0), (b + 1, E))
         for b, w in enumerate(withins)], axis=0)
    counts = (lax.slice(boffs, (nb - 1, 0), (nb, E))
              + lax.slice(bs, (nb - 1, 0), (nb, E)))
    padded = jnp.floor((counts + (TILE - 1.0)) * (1.0 / TILE)) * TILE
    m8 = (lax.broadcasted_iota(jnp.int32, (E, E), 0)
          < lax.broadcasted_iota(jnp.int32, (E, E), 1)).astype(jnp.float32)
    offs = jnp.dot(padded, m8, preferred_element_type=jnp.float32)
    posf = offs + excl
    p0 = jnp.sum(oh1 * posf, axis=1, keepdims=True)
    p1 = jnp.sum(oh2 * posf, axis=1, keepdims=True)
    pos_ref[...] = jnp.concatenate([p0, p1], axis=1).astype(jnp.int32)
    it = lax.broadcasted_iota(jnp.int32, (48, E), 0).astype(jnp.float32) * TILE
    te = jnp.sum((offs <= it).astype(jnp.int32), axis=1, keepdims=True) - 1
    te_ref[...] = jnp.clip(te, 0, E - 1)


def _router(flat, gate_w, gate_b):
    return pl.pallas_call(
        _router_body,
        out_shape=[
            jax.ShapeDtypeStruct((N, E), jnp.float32),
            jax.ShapeDtypeStruct((N, 2), jnp.int32),
            jax.ShapeDtypeStruct((48, 1), jnp.int32),
            jax.ShapeDtypeStruct((N, 2), jnp.float32),
        ],
    )(flat, gate_w, gate_b)


def _dispatch_body(flat_hbm, posr_hbm, wrows_hbm, xs_hbm, wtab_hbm,
                   idx_v, xbuf, wrv, semx0, semx1, semw):
    wid = lax.axis_index("s") * 2 + lax.axis_index("c")
    pltpu.sync_copy(posr_hbm.at[wid], idx_v)
    pltpu.sync_copy(wrows_hbm.at[wid], wrv)
    semx = [semx0, semx1]
    nch = CHT // SUB
    pltpu.sync_copy(flat_hbm.at[pl.ds(wid * CHT, SUB)], xbuf.at[0])
    pend = [None] * nch
    for c in range(nch):
        bi = c % 2
        pend[c] = (
            pltpu.async_copy(xbuf.at[bi], xs_hbm.at[idx_v.at[c]], semx[bi]),
            pltpu.async_copy(xbuf.at[bi], xs_hbm.at[idx_v.at[4 + c]],
                             semx[bi]),
            pltpu.async_copy(wrv.at[c], wtab_hbm.at[idx_v.at[c]], semw),
            pltpu.async_copy(wrv.at[4 + c], wtab_hbm.at[idx_v.at[4 + c]],
                             semw))
        if c + 1 < nch:
            if c - 1 >= 0:
                pend[c - 1][0].wait()
                pend[c - 1][1].wait()
            pltpu.sync_copy(flat_hbm.at[pl.ds(wid * CHT + (c + 1) * SUB, SUB)],
                            xbuf.at[(c + 1) % 2])
    for c in (nch - 2, nch - 1):
        for cp in pend[c][:2]:
            cp.wait()
    for c in range(nch):
        pend[c][2].wait()
        pend[c][3].wait()


def _dispatch(flat, posr, wrows):
    mesh = plsc.VectorSubcoreMesh(core_axis_name="c", subcore_axis_name="s")
    return pl.kernel(
        _dispatch_body,
        out_type=[
            jax.ShapeDtypeStruct((PADDED, D), jnp.float32),
            jax.ShapeDtypeStruct((PADDED, 128), jnp.float32),
        ],
        mesh=mesh,
        scratch_types=[
            pltpu.VMEM((8, SUB), jnp.int32),
            pltpu.VMEM((2, SUB, D), jnp.float32),
            pltpu.VMEM((8, SUB, 128), jnp.float32),
            pltpu.SemaphoreType.DMA,
            pltpu.SemaphoreType.DMA,
            pltpu.SemaphoreType.DMA,
        ],
    )(flat, posr, wrows)


def _mlp_body(te_ref, x_ref, w1_hbm, b1_ref, w2_hbm, b2_ref, wt_ref, y_ref,
              w1v, w2v, sem1, sem2, sem3, sem4):
    i = pl.program_id(0)
    e = te_ref[i]
    eprev = te_ref[jnp.maximum(i - 1, 0)]
    change = jnp.logical_or(i == 0, e != eprev)
    q = F // 4
    sems = [sem1, sem2, sem3, sem4]

    def w_copies(k):
        sl = pl.ds(k * q, q)
        return (pltpu.make_async_copy(w1_hbm.at[e, :, sl], w1v.at[:, sl],
                                      sems[k]),
                pltpu.make_async_copy(w2_hbm.at[e, sl, :], w2v.at[sl, :],
                                      sems[k]))

    @pl.when(change)
    def _():
        for k in range(4):
            c1, c2 = w_copies(k)
            c1.start()
            c2.start()

    x = x_ref[...]
    acc = None
    for j in range(NJ):
        if j % 2 == 0:
            @pl.when(change)
            def _():
                c1, c2 = w_copies(j // 2)
                c1.wait()
                c2.wait()
        sl = pl.ds(j * FT, FT)
        h = jnp.maximum(
            jnp.dot(x, w1v[:, sl], preferred_element_type=jnp.float32)
            + b1_ref[0, :, sl], 0.0)
        part = jnp.dot(h, w2v[sl, :], preferred_element_type=jnp.float32)
        acc = part if acc is None else acc + part
    y_ref[...] = (acc + b2_ref[0]) * wt_ref[:, 0:1]


def _mlp(texp, xs, w1, b1, w2, b2, wtab):
    grid_spec = pltpu.PrefetchScalarGridSpec(
        num_scalar_prefetch=1,
        grid=(NT,),
        in_specs=[
            pl.BlockSpec((TILE, D), lambda i, s: (i, 0)),
            pl.BlockSpec(memory_space=pl.ANY),
            pl.BlockSpec((1, 1, F), lambda i, s: (s[i], 0, 0)),
            pl.BlockSpec(memory_space=pl.ANY),
            pl.BlockSpec((1, 1, D), lambda i, s: (s[i], 0, 0)),
            pl.BlockSpec((TILE, 128), lambda i, s: (i, 0)),
        ],
        out_specs=pl.BlockSpec((TILE, D), lambda i, s: (i, 0)),
        scratch_shapes=[
            pltpu.VMEM((D, F), jnp.float32),
            pltpu.VMEM((F, D), jnp.float32),
            pltpu.SemaphoreType.DMA,
            pltpu.SemaphoreType.DMA,
            pltpu.SemaphoreType.DMA,
            pltpu.SemaphoreType.DMA,
        ],
    )
    return pl.pallas_call(
        _mlp_body,
        grid_spec=grid_spec,
        out_shape=jax.ShapeDtypeStruct((PADDED, D), jnp.float32),
        compiler_params=pltpu.CompilerParams(
            dimension_semantics=("arbitrary",),
            vmem_limit_bytes=60 * 1024 * 1024),
    )(texp, xs, w1, b1.reshape(E, 1, F), w2, b2.reshape(E, 1, D), wtab)


SUBC = 16
NCHC = CHT // SUBC


def _combine_body(ys_hbm, posr_hbm, out_hbm,
                  idx_v, y0, y1, ob, semg0, semg1, sems0, sems1):
    wid = lax.axis_index("s") * 2 + lax.axis_index("c")
    pltpu.sync_copy(posr_hbm.at[wid], idx_v)
    semg = [semg0, semg1]
    sems = [sems0, sems1]

    def fire(c):
        bi = c % 2
        return (pltpu.async_copy(ys_hbm.at[idx_v.at[c]], y0.at[bi], semg[bi]),
                pltpu.async_copy(ys_hbm.at[idx_v.at[NCHC + c]], y1.at[bi],
                                 semg[bi]))

    gath = [None] * NCHC
    stores = [None] * NCHC
    gath[0] = fire(0)
    for c in range(NCHC):
        bi = c % 2
        if c + 1 < NCHC:
            gath[c + 1] = fire(c + 1)
        gath[c][0].wait()
        gath[c][1].wait()

        def tok_body(t, _):
            for qq in range(D // 16):
                sl = pl.ds(16 * qq, 16)
                ob[bi, t, sl] = y0[bi, t, sl] + y1[bi, t, sl]
            return 0

        if c - 2 >= 0:
            stores[c - 2].wait()
        lax.fori_loop(0, SUBC, tok_body, 0)
        stores[c] = pltpu.async_copy(
            ob.at[bi], out_hbm.at[pl.ds(wid * CHT + c * SUBC, SUBC)],
            sems[bi])
    stores[NCHC - 2].wait()
    stores[NCHC - 1].wait()


def _combine(ys, posrc):
    mesh = plsc.VectorSubcoreMesh(core_axis_name="c", subcore_axis_name="s")
    return pl.kernel(
        _combine_body,
        out_type=jax.ShapeDtypeStruct((N, D), jnp.float32),
        mesh=mesh,
        scratch_types=[
            pltpu.VMEM((2 * NCHC, SUBC), jnp.int32),
            pltpu.VMEM((2, SUBC, D), jnp.float32),
            pltpu.VMEM((2, SUBC, D), jnp.float32),
            pltpu.VMEM((2, SUBC, D), jnp.float32),
            pltpu.SemaphoreType.DMA,
            pltpu.SemaphoreType.DMA,
            pltpu.SemaphoreType.DMA,
            pltpu.SemaphoreType.DMA,
        ],
    )(ys, posrc)


def kernel(hidden_states, gate_w, gate_b, w1, b1, w2, b2):
    seq, bsz, d = hidden_states.shape
    flat = hidden_states.reshape(-1, d)
    logits, pos2, te48, wv = _router(flat, gate_w, gate_b.reshape(1, E))
    texp = te48.reshape(-1)[:NT]
    posr = (pos2.T.reshape(2, NW, CHT // SUB, SUB)
            .transpose(1, 0, 2, 3).reshape(NW, 8, SUB))
    posrc = (pos2.T.reshape(2, NW, NCHC, SUBC)
             .transpose(1, 0, 2, 3).reshape(NW, 2 * NCHC, SUBC))
    wrows = jnp.broadcast_to(
        (wv.T.reshape(2, NW, CHT // SUB, SUB)
         .transpose(1, 0, 2, 3).reshape(NW, 8, SUB))[..., None],
        (NW, 8, SUB, 128))
    xs, wtab = _dispatch(flat, posr, wrows)
    ys = _mlp(texp, xs, w1, b1, w2, b2, wtab)
    final = _combine(ys, posrc)
    return final.reshape(seq, bsz, d), logits


# speedup vs baseline: 1.1581x; 1.1454x over previous
import functools

import jax
import jax.numpy as jnp
from jax import lax
from jax.experimental import pallas as pl
from jax.experimental.pallas import tpu as pltpu
from jax.experimental.pallas import tpu_sc as plsc

N = 4096
D = 1024
F = 4096
E = 8
TILE = 256
PADDED = 10240
NT = PADDED // TILE
FT = 512
NJ = F // FT
NW = 32
CHT = N // NW
SUB = 32


def _router_body(x_ref, gw_ref, gb_ref, lg_ref, pos_ref, te_ref, wv_ref):
    x = x_ref[...]
    lg = jnp.dot(x, gw_ref[...], preferred_element_type=jnp.float32) + gb_ref[...]
    lg_ref[...] = lg
    mx = jnp.max(lg, axis=1, keepdims=True)
    ex = jnp.exp(lg - mx)
    sm = ex / jnp.sum(ex, axis=1, keepdims=True)
    io = lax.broadcasted_iota(jnp.int32, sm.shape, 1)
    m1 = jnp.max(sm, axis=1, keepdims=True)
    i1 = jnp.min(jnp.where(sm == m1, io, E), axis=1, keepdims=True)
    oh1 = (io == i1).astype(jnp.float32)
    sm2 = jnp.where(io == i1, -1.0, sm)
    m2 = jnp.max(sm2, axis=1, keepdims=True)
    i2 = jnp.min(jnp.where(sm2 == m2, io, E), axis=1, keepdims=True)
    oh2 = (io == i2).astype(jnp.float32)
    den = m1 + m2
    wv_ref[...] = jnp.concatenate([m1 / den, m2 / den], axis=1)
    c = oh1 + oh2
    nb = N // 128
    tri = (lax.broadcasted_iota(jnp.int32, (128, 128), 0)
           > lax.broadcasted_iota(jnp.int32, (128, 128), 1)).astype(jnp.float32)
    withins = [jnp.dot(tri, lax.slice(c, (b * 128, 0), ((b + 1) * 128, E)),
                       preferred_element_type=jnp.float32) for b in range(nb)]
    bs = jnp.concatenate(
        [lax.slice(w, (127, 0), (128, E))
         + lax.slice(c, (b * 128 + 127, 0), (b * 128 + 128, E))
         for b, w in enumerate(withins)], axis=0)
    tri_b = (lax.broadcasted_iota(jnp.int32, (nb, nb), 0)
             > lax.broadcasted_iota(jnp.int32, (nb, nb), 1)).astype(jnp.float32)
    boffs = jnp.dot(tri_b, bs, preferred_element_type=jnp.float32)
    excl = jnp.concatenate(
        [w + lax.slice(boffs, (b, 0), (b + 1, E))
         for b, w in enumerate(withins)], axis=0)
    counts = (lax.slice(boffs, (nb - 1, 0), (nb, E))
              + lax.slice(bs, (nb - 1, 0), (nb, E)))
    padded = jnp.floor((counts + (TILE - 1.0)) * (1.0 / TILE)) * TILE
    m8 = (lax.broadcasted_iota(jnp.int32, (E, E), 0)
          < lax.broadcasted_iota(jnp.int32, (E, E), 1)).astype(jnp.float32)
    offs = jnp.dot(padded, m8, preferred_element_type=jnp.float32)
    posf = offs + excl
    p0 = jnp.sum(oh1 * posf, axis=1, keepdims=True)
    p1 = jnp.sum(oh2 * posf, axis=1, keepdims=True)
    pos_ref[...] = jnp.concatenate([p0, p1], axis=1).astype(jnp.int32)
    it = lax.broadcasted_iota(jnp.int32, (48, E), 0).astype(jnp.float32) * TILE
    te = jnp.clip(jnp.sum((offs <= it).astype(jnp.int32), axis=1,
                          keepdims=True) - 1, 0, E - 1)
    teF = te.astype(jnp.float32)
    io8 = lax.broadcasted_iota(jnp.int32, (48, E), 1).astype(jnp.float32)
    used = (padded > 0.0).astype(jnp.float32)
    ohte = (io8 == teF).astype(jnp.float32)
    usedte = jnp.sum(ohte * used, axis=1, keepdims=True)
    startf = jnp.sum(ohte * offs, axis=1, keepdims=True)
    endf = jnp.sum(ohte * (offs + padded), axis=1, keepdims=True) - TILE
    it0 = lax.broadcasted_iota(jnp.int32, (48, 1), 0).astype(jnp.float32) * TILE
    change = ((it0 == startf) * usedte)
    is_last = ((it0 == endf) * usedte)
    nexte = jnp.clip(jnp.min(
        jnp.where((io8 > teF) * used > 0.0, io8, 8.0), axis=1, keepdims=True),
        0.0, 8.0)
    ks = jnp.sum(((io8 <= teF) * used), axis=1, keepdims=True) - 1.0

    def mod3(v):
        return v - jnp.floor(v * (1.0 / 3.0)) * 3.0

    b0 = mod3(2.0 * ks)
    b1 = mod3(2.0 * ks + 1.0)
    bD = mod3(2.0 * ks + 2.0)
    first = (it0 == 0.0).astype(jnp.float32)
    eight = jnp.full_like(teF, 8.0)
    aA = jnp.where(first > 0.0, teF, eight)
    aB = jnp.where(first > 0.0, teF, eight)
    cC = jnp.where(is_last > 0.0, nexte, eight)
    cD = jnp.where(change > 0.0, nexte, eight)
    zero = jnp.zeros_like(teF)
    cols = [te.astype(jnp.float32), b0, b1, aA, aB, cC, b0, cD, bD, change,
            zero, zero, zero, zero, zero, zero]
    te_ref[...] = jnp.concatenate(cols, axis=1).astype(jnp.int32)


def _router(flat, gate_w, gate_b):
    return pl.pallas_call(
        _router_body,
        out_shape=[
            jax.ShapeDtypeStruct((N, E), jnp.float32),
            jax.ShapeDtypeStruct((N, 2), jnp.int32),
            jax.ShapeDtypeStruct((48, 16), jnp.int32),
            jax.ShapeDtypeStruct((N, 2), jnp.float32),
        ],
    )(flat, gate_w, gate_b)


def _dispatch_body(flat_hbm, posr_hbm, wrows_hbm, xs_hbm, wtab_hbm,
                   idx_v, xbuf, wrv, semx0, semx1, semw):
    wid = lax.axis_index("s") * 2 + lax.axis_index("c")
    pltpu.sync_copy(posr_hbm.at[wid], idx_v)
    pltpu.sync_copy(wrows_hbm.at[wid], wrv)
    semx = [semx0, semx1]
    nch = CHT // SUB
    pltpu.sync_copy(flat_hbm.at[pl.ds(wid * CHT, SUB)], xbuf.at[0])
    pend = [None] * nch
    for c in range(nch):
        bi = c % 2
        pend[c] = (
            pltpu.async_copy(xbuf.at[bi], xs_hbm.at[idx_v.at[c]], semx[bi]),
            pltpu.async_copy(xbuf.at[bi], xs_hbm.at[idx_v.at[4 + c]],
                             semx[bi]),
            pltpu.async_copy(wrv.at[c], wtab_hbm.at[idx_v.at[c]], semw),
            pltpu.async_copy(wrv.at[4 + c], wtab_hbm.at[idx_v.at[4 + c]],
                             semw))
        if c + 1 < nch:
            if c - 1 >= 0:
                pend[c - 1][0].wait()
                pend[c - 1][1].wait()
            pltpu.sync_copy(flat_hbm.at[pl.ds(wid * CHT + (c + 1) * SUB, SUB)],
                            xbuf.at[(c + 1) % 2])
    for c in (nch - 2, nch - 1):
        for cp in pend[c][:2]:
            cp.wait()
    for c in range(nch):
        pend[c][2].wait()
        pend[c][3].wait()


def _dispatch(flat, posr, wrows):
    mesh = plsc.VectorSubcoreMesh(core_axis_name="c", subcore_axis_name="s")
    return pl.kernel(
        _dispatch_body,
        out_type=[
            jax.ShapeDtypeStruct((PADDED, D), jnp.float32),
            jax.ShapeDtypeStruct((PADDED, 128), jnp.float32),
        ],
        mesh=mesh,
        scratch_types=[
            pltpu.VMEM((8, SUB), jnp.int32),
            pltpu.VMEM((2, SUB, D), jnp.float32),
            pltpu.VMEM((8, SUB, 128), jnp.float32),
            pltpu.SemaphoreType.DMA,
            pltpu.SemaphoreType.DMA,
            pltpu.SemaphoreType.DMA,
        ],
    )(flat, posr, wrows)


F2 = F // 2


def _mlp_body(sc_ref, x_ref, w1_hbm, b1_ref, w2_hbm, b2_ref, wt_ref, y_ref,
              w1h, w2h, semA, semB, semC):
    i = pl.program_id(0)
    sems = [semA, semB, semC]

    def s(col):
        return sc_ref[i * 16 + col]

    def fire(e, half, buf):
        @pl.when(e < E)
        def _():
            for b in range(3):
                @pl.when(buf == b)
                def _():
                    sl = pl.ds(half * F2, F2)
                    pltpu.make_async_copy(
                        w1_hbm.at[e, :, sl], w1h.at[b], sems[b]).start()
                    pltpu.make_async_copy(
                        w2_hbm.at[e, sl, :], w2h.at[b], sems[b]).start()

    def wait_buf(buf):
        for b in range(3):
            @pl.when(buf == b)
            def _():
                pltpu.make_async_copy(
                    w1_hbm.at[0, :, pl.ds(0, F2)], w1h.at[b], sems[b]).wait()
                pltpu.make_async_copy(
                    w2_hbm.at[0, pl.ds(0, F2), :], w2h.at[b], sems[b]).wait()

    fire(s(3), 0, s(1))
    fire(s(4), 1, s(2))
    fire(s(7), 0, s(8))
    b0 = s(1)
    b1 = s(2)

    @pl.when(s(9) == 1)
    def _():
        wait_buf(b0)

    x = x_ref[...]
    acc = None
    for j in range(NJ // 2):
        sl = pl.ds(j * FT, FT)
        gsl = pl.ds(j * FT, FT)
        h = jnp.maximum(
            jnp.dot(x, w1h[b0, :, sl], preferred_element_type=jnp.float32)
            + b1_ref[0, :, gsl], 0.0)
        part = jnp.dot(h, w2h[b0, sl, :], preferred_element_type=jnp.float32)
        acc = part if acc is None else acc + part

    fire(s(5), 1, s(6))

    @pl.when(s(9) == 1)
    def _():
        wait_buf(b1)

    for j in range(NJ // 2):
        sl = pl.ds(j * FT, FT)
        gsl = pl.ds(F2 + j * FT, FT)
        h = jnp.maximum(
            jnp.dot(x, w1h[b1, :, sl], preferred_element_type=jnp.float32)
            + b1_ref[0, :, gsl], 0.0)
        acc = acc + jnp.dot(h, w2h[b1, sl, :],
                            preferred_element_type=jnp.float32)
    y_ref[...] = (acc + b2_ref[0]) * wt_ref[:, 0:1]


def _mlp(sched, xs, w1, b1, w2, b2, wtab):
    grid_spec = pltpu.PrefetchScalarGridSpec(
        num_scalar_prefetch=1,
        grid=(NT,),
        in_specs=[
            pl.BlockSpec((TILE, D), lambda i, s: (i, 0)),
            pl.BlockSpec(memory_space=pl.ANY),
            pl.BlockSpec((1, 1, F), lambda i, s: (s[i * 16], 0, 0)),
            pl.BlockSpec(memory_space=pl.ANY),
            pl.BlockSpec((1, 1, D), lambda i, s: (s[i * 16], 0, 0)),
            pl.BlockSpec((TILE, 128), lambda i, s: (i, 0)),
        ],
        out_specs=pl.BlockSpec((TILE, D), lambda i, s: (i, 0)),
        scratch_shapes=[
            pltpu.VMEM((3, D, F2), jnp.float32),
            pltpu.VMEM((3, F2, D), jnp.float32),
            pltpu.SemaphoreType.DMA,
            pltpu.SemaphoreType.DMA,
            pltpu.SemaphoreType.DMA,
        ],
    )
    return pl.pallas_call(
        _mlp_body,
        grid_spec=grid_spec,
        out_shape=jax.ShapeDtypeStruct((PADDED, D), jnp.float32),
        compiler_params=pltpu.CompilerParams(
            dimension_semantics=("arbitrary",),
            vmem_limit_bytes=60 * 1024 * 1024),
    )(sched, xs, w1, b1.reshape(E, 1, F), w2, b2.reshape(E, 1, D), wtab)


SUBC = 16
NCHC = CHT // SUBC


def _combine_body(ys_hbm, posr_hbm, out_hbm,
                  idx_v, y0, y1, ob, semg0, semg1, sems0, sems1):
    wid = lax.axis_index("s") * 2 + lax.axis_index("c")
    pltpu.sync_copy(posr_hbm.at[wid], idx_v)
    semg = [semg0, semg1]
    sems = [sems0, sems1]

    def fire(c):
        bi = c % 2
        return (pltpu.async_copy(ys_hbm.at[idx_v.at[c]], y0.at[bi], semg[bi]),
                pltpu.async_copy(ys_hbm.at[idx_v.at[NCHC + c]], y1.at[bi],
                                 semg[bi]))

    gath = [None] * NCHC
    stores = [None] * NCHC
    gath[0] = fire(0)
    for c in range(NCHC):
        bi = c % 2
        if c + 1 < NCHC:
            gath[c + 1] = fire(c + 1)
        gath[c][0].wait()
        gath[c][1].wait()

        def tok_body(t, _):
            for qq in range(D // 16):
                sl = pl.ds(16 * qq, 16)
                ob[bi, t, sl] = y0[bi, t, sl] + y1[bi, t, sl]
            return 0

        if c - 2 >= 0:
            stores[c - 2].wait()
        lax.fori_loop(0, SUBC, tok_body, 0)
        stores[c] = pltpu.async_copy(
            ob.at[bi], out_hbm.at[pl.ds(wid * CHT + c * SUBC, SUBC)],
            sems[bi])
    stores[NCHC - 2].wait()
    stores[NCHC - 1].wait()


def _combine(ys, posrc):
    mesh = plsc.VectorSubcoreMesh(core_axis_name="c", subcore_axis_name="s")
    return pl.kernel(
        _combine_body,
        out_type=jax.ShapeDtypeStruct((N, D), jnp.float32),
        mesh=mesh,
        scratch_types=[
            pltpu.VMEM((2 * NCHC, SUBC), jnp.int32),
            pltpu.VMEM((2, SUBC, D), jnp.float32),
            pltpu.VMEM((2, SUBC, D), jnp.float32),
            pltpu.VMEM((2, SUBC, D), jnp.float32),
            pltpu.SemaphoreType.DMA,
            pltpu.SemaphoreType.DMA,
            pltpu.SemaphoreType.DMA,
            pltpu.SemaphoreType.DMA,
        ],
    )(ys, posrc)


def kernel(hidden_states, gate_w, gate_b, w1, b1, w2, b2):
    seq, bsz, d = hidden_states.shape
    flat = hidden_states.reshape(-1, d)
    logits, pos2, sched, wv = _router(flat, gate_w, gate_b.reshape(1, E))
    sched_flat = sched.reshape(-1)
    posr = (pos2.T.reshape(2, NW, CHT // SUB, SUB)
            .transpose(1, 0, 2, 3).reshape(NW, 8, SUB))
    posrc = (pos2.T.reshape(2, NW, NCHC, SUBC)
             .transpose(1, 0, 2, 3).reshape(NW, 2 * NCHC, SUBC))
    wrows = jnp.broadcast_to(
        (wv.T.reshape(2, NW, CHT // SUB, SUB)
         .transpose(1, 0, 2, 3).reshape(NW, 8, SUB))[..., None],
        (NW, 8, SUB, 128))
    xs, wtab = _dispatch(flat, posr, wrows)
    ys = _mlp(sched_flat, xs, w1, b1, w2, b2, wtab)
    final = _combine(ys, posrc)
    return final.reshape(seq, bsz, d), logits
